# Initial kernel scaffold; baseline (speedup 1.0000x reference)
import jax
import jax.numpy as jnp
from jax.experimental import pallas as pl


def kernel(x, edge_index, edge_prob, W_proj, W_tp, a_src, a_trg, a_tp, W_skip, bias):
    raise NotImplementedError("write your pallas kernel here")


# speedup vs baseline: 34.7480x; 34.7480x over previous
import functools

import jax
import jax.numpy as jnp
from jax import lax
from jax.experimental import pallas as pl
from jax.experimental.pallas import tpu as pltpu
from jax.experimental.pallas import tpu_sc as plsc

N = 10000
E = 320000
D = 128
H = 8
F = 16
HF = H * F
HH = HF // 2

NW = 32
EPT = E // NW
K = 80
NCH = EPT // K
STRIPE = 624
ZCH = 208
TAIL = N - 16 * STRIPE
BN = 400


def _dense_body(x_ref, wpt0_ref, wpt1_ref, wst_ref, asrc_ref, atrg_ref,
                ra_ref, proj0_ref, proj1_ref, skip_ref, srctab_ref,
                trgtab_ref):
    xb = x_ref[...]
    p0 = jnp.dot(xb, wpt0_ref[...], preferred_element_type=jnp.float32)
    p1 = jnp.dot(xb, wpt1_ref[...], preferred_element_type=jnp.float32)
    proj0_ref[...] = p0
    proj1_ref[...] = p1
    skip_ref[...] = jnp.dot(xb, wst_ref[...], preferred_element_type=jnp.float32)
    asrc = asrc_ref[...]
    atrg = atrg_ref[...]
    ra = ra_ref[...]
    srctab_ref[...] = (
        jnp.dot(p0 * asrc[:, :HH], ra[:HH], preferred_element_type=jnp.float32)
        + jnp.dot(p1 * asrc[:, HH:], ra[HH:], preferred_element_type=jnp.float32))
    trgtab_ref[...] = (
        jnp.dot(p0 * atrg[:, :HH], ra[:HH], preferred_element_type=jnp.float32)
        + jnp.dot(p1 * atrg[:, HH:], ra[HH:], preferred_element_type=jnp.float32))


def _dense_stage(x, wpt0, wpt1, wst, asrc, atrg, ra):
    grid = (N // BN,)
    return pl.pallas_call(
        _dense_body,
        grid=grid,
        in_specs=[
            pl.BlockSpec((BN, D), lambda i: (i, 0)),
            pl.BlockSpec((D, HH), lambda i: (0, 0)),
            pl.BlockSpec((D, HH), lambda i: (0, 0)),
            pl.BlockSpec((D, HF), lambda i: (0, 0)),
            pl.BlockSpec((1, HF), lambda i: (0, 0)),
            pl.BlockSpec((1, HF), lambda i: (0, 0)),
            pl.BlockSpec((HF, 16), lambda i: (0, 0)),
        ],
        out_specs=[
            pl.BlockSpec((BN, HH), lambda i: (i, 0)),
            pl.BlockSpec((BN, HH), lambda i: (i, 0)),
            pl.BlockSpec((BN, HF), lambda i: (i, 0)),
            pl.BlockSpec((BN, 16), lambda i: (i, 0)),
            pl.BlockSpec((BN, 16), lambda i: (i, 0)),
        ],
        out_shape=[
            jax.ShapeDtypeStruct((N, HH), jnp.float32),
            jax.ShapeDtypeStruct((N, HH), jnp.float32),
            jax.ShapeDtypeStruct((N, HF), jnp.float32),
            jax.ShapeDtypeStruct((N, 16), jnp.float32),
            jax.ShapeDtypeStruct((N, 16), jnp.float32),
        ],
    )(x, wpt0, wpt1, wst, asrc, atrg, ra)


def _sc_edge_body(srctab_hbm, trgtab_hbm, proj0_hbm, proj1_hbm, src_hbm,
                  trg3_hbm, prob_hbm, c_hbm,
                  denom_hbm, numa_hbm, numb_hbm,
                  src_v, trg2_v, prob_v, a_v, b_v, p_v, e_v, w_v, c_v,
                  z64_v, z16_v, denom_sh, numer_sh, sem0, sem1, sem2):
    cid = lax.axis_index("c")
    sid = lax.axis_index("s")
    wid = sid * 2 + cid
    base = wid * EPT

    pltpu.sync_copy(src_hbm.at[pl.ds(base, EPT)], src_v)
    pltpu.sync_copy(trg3_hbm.at[wid], trg2_v)
    pltpu.sync_copy(prob_hbm.at[pl.ds(base, EPT)], prob_v.at[pl.ds(0, EPT)])
    pltpu.sync_copy(c_hbm, c_v)
    c_vec = c_v[...]

    zv = jnp.zeros((16,), jnp.float32)

    def zfill(r, _):
        for h in range(4):
            z64_v[r, pl.ds(h * 16, 16)] = zv
        z16_v[r, :] = zv
        return 0

    lax.fori_loop(0, ZCH, zfill, 0)

    def zero_stripes(zero_denom):
        for kk in range(STRIPE // ZCH):
            r0 = sid * STRIPE + kk * ZCH
            pltpu.sync_copy(z64_v, numer_sh.at[pl.ds(r0, ZCH)])
            if zero_denom:
                pltpu.sync_copy(z16_v, denom_sh.at[pl.ds(r0, ZCH)])

        @pl.when(sid == 15)
        def _zero_tail():
            pltpu.sync_copy(z64_v.at[pl.ds(0, TAIL)],
                            numer_sh.at[pl.ds(16 * STRIPE, TAIL)])
            if zero_denom:
                pltpu.sync_copy(z16_v.at[pl.ds(0, TAIL)],
                                denom_sh.at[pl.ds(16 * STRIPE, TAIL)])

    zero_stripes(True)
    plsc.subcore_barrier()

    def run_pass(proj_hbm, h0, scatter_denom):
        def chunk(j, _):
            src_slice = src_v.at[pl.ds(j * K, K)]
            trg_row = trg2_v.at[j]
            ca = pltpu.async_copy(srctab_hbm.at[src_slice], a_v, sem0)
            cb = pltpu.async_copy(trgtab_hbm.at[trg_row], b_v, sem1)
            cp = pltpu.async_copy(proj_hbm.at[src_slice], p_v, sem2)
            ca.wait()
            cb.wait()
            cp.wait()

            def edge(i, _):
                a = a_v[i, :]
                b = b_v[i, :]
                pv = prob_v[pl.ds(j * K + i, 16)]
                s = a + b + pv[0] * c_vec
                s = jnp.where(s > 0.0, s, 0.2 * s)
                e = jnp.exp(s)
                if scatter_denom:
                    e_v[i, :] = e
                for h in range(4):
                    w_v[i, pl.ds(h * 16, 16)] = (
                        p_v[i, pl.ds(h * 16, 16)] * e[h0 + h])
                return 0

            lax.fori_loop(0, K, edge, 0)
            if scatter_denom:
                pltpu.sync_copy(e_v, denom_sh.at[trg_row], add=True)
            pltpu.sync_copy(w_v, numer_sh.at[trg_row], add=True)
            return 0

        lax.fori_loop(0, NCH, chunk, 0)

    def writeback(src_sh, dst_hbm, width_tail_buf):
        out_base = cid * N + sid * STRIPE
        pltpu.sync_copy(src_sh.at[pl.ds(sid * STRIPE, STRIPE)],
                        dst_hbm.at[pl.ds(out_base, STRIPE)])

        @pl.when(sid == 15)
        def _tail():
            pltpu.sync_copy(src_sh.at[pl.ds(16 * STRIPE, TAIL)],
                            dst_hbm.at[pl.ds(cid * N + 16 * STRIPE, TAIL)])

    run_pass(proj0_hbm, 0, True)
    plsc.subcore_barrier()
    writeback(denom_sh, denom_hbm, z16_v)
    writeback(numer_sh, numa_hbm, z64_v)
    zero_stripes(False)
    plsc.subcore_barrier()

    run_pass(proj1_hbm, 4, False)
    plsc.subcore_barrier()
    writeback(numer_sh, numb_hbm, z64_v)


def _sc_edge_stage(srctab, trgtab, proj0, proj1, src, trg3, prob, cvec):
    mesh = plsc.VectorSubcoreMesh(core_axis_name="c", subcore_axis_name="s")
    fn = pl.kernel(
        _sc_edge_body,
        compiler_params=pltpu.CompilerParams(use_tc_tiling_on_sc=False),
        out_type=[
            jax.ShapeDtypeStruct((2 * N, 16), jnp.float32),
            jax.ShapeDtypeStruct((2 * N, HH), jnp.float32),
            jax.ShapeDtypeStruct((2 * N, HH), jnp.float32),
        ],
        mesh=mesh,
        scratch_types=[
            pltpu.VMEM((EPT,), jnp.int32),
            pltpu.VMEM((NCH, K), jnp.int32),
            pltpu.VMEM((EPT + 16,), jnp.float32),
            pltpu.VMEM((K, 16), jnp.float32),
            pltpu.VMEM((K, 16), jnp.float32),
            pltpu.VMEM((K, HH), jnp.float32),
            pltpu.VMEM((K, 16), jnp.float32),
            pltpu.VMEM((K, HH), jnp.float32),
            pltpu.VMEM((16,), jnp.float32),
            pltpu.VMEM((ZCH, HH), jnp.float32),
            pltpu.VMEM((ZCH, 16), jnp.float32),
            pltpu.VMEM_SHARED((N, 16), jnp.float32),
            pltpu.VMEM_SHARED((N, HH), jnp.float32),
            pltpu.SemaphoreType.DMA,
            pltpu.SemaphoreType.DMA,
            pltpu.SemaphoreType.DMA,
        ],
    )
    return fn(srctab, trgtab, proj0, proj1, src, trg3, prob, cvec)


def _combine_body(na0_ref, na1_ref, nb0_ref, nb1_ref, d0_ref, d1_ref,
                  skip_ref, bias_ref, r_ref, out_ref):
    d = jnp.dot(d0_ref[...] + d1_ref[...], r_ref[...],
                preferred_element_type=jnp.float32) + 1e-16
    num = jnp.concatenate([na0_ref[...] + na1_ref[...],
                           nb0_ref[...] + nb1_ref[...]], axis=-1)
    y = num / d + skip_ref[...] + bias_ref[...]
    out_ref[...] = jnp.where(y > 0.0, y, jnp.exp(jnp.minimum(y, 0.0)) - 1.0)


def _combine_stage(na0, na1, nb0, nb1, d0, d1, skip, bias, r):
    grid = (N // BN,)
    return pl.pallas_call(
        _combine_body,
        grid=grid,
        in_specs=[
            pl.BlockSpec((BN, HH), lambda i: (i, 0)),
            pl.BlockSpec((BN, HH), lambda i: (i, 0)),
            pl.BlockSpec((BN, HH), lambda i: (i, 0)),
            pl.BlockSpec((BN, HH), lambda i: (i, 0)),
            pl.BlockSpec((BN, 16), lambda i: (i, 0)),
            pl.BlockSpec((BN, 16), lambda i: (i, 0)),
            pl.BlockSpec((BN, HF), lambda i: (i, 0)),
            pl.BlockSpec((1, HF), lambda i: (0, 0)),
            pl.BlockSpec((16, HF), lambda i: (0, 0)),
        ],
        out_specs=pl.BlockSpec((BN, HF), lambda i: (i, 0)),
        out_shape=jax.ShapeDtypeStruct((N, HF), jnp.float32),
    )(na0, na1, nb0, nb1, d0, d1, skip, bias, r)


def kernel(x, edge_index, edge_prob, W_proj, W_tp, a_src, a_trg, a_tp,
           W_skip, bias):
    wpt = W_proj.T
    wst = W_skip.T
    asrc = a_src.reshape(1, HF)
    atrg = a_trg.reshape(1, HF)
    hf_i = lax.broadcasted_iota(jnp.int32, (HF, 16), 0)
    col_i = lax.broadcasted_iota(jnp.int32, (HF, 16), 1)
    ra = (hf_i // F == col_i).astype(jnp.float32)
    row_i = lax.broadcasted_iota(jnp.int32, (16, HF), 0)
    hf_j = lax.broadcasted_iota(jnp.int32, (16, HF), 1)
    r = (row_i == hf_j // F).astype(jnp.float32)
    c8 = (W_tp[:, 0] * a_tp.reshape(HF)).reshape(H, F).sum(-1)
    c16 = jnp.concatenate([c8, c8]).astype(jnp.float32)

    proj0, proj1, skip, srctab, trgtab = _dense_stage(
        x, wpt[:, :HH], wpt[:, HH:], wst, asrc, atrg, ra)

    src = edge_index[0]
    trg3 = edge_index[1].reshape(NW, NCH, K)
    prob = edge_prob[:, 0]
    denom, numa, numb = _sc_edge_stage(srctab, trgtab, proj0, proj1, src,
                                       trg3, prob, c16)

    out = _combine_stage(numa[:N], numa[N:], numb[:N], numb[N:],
                         denom[:N], denom[N:], skip, bias.reshape(1, HF), r)
    return (out, edge_index, edge_prob)


# speedup vs baseline: 108.8849x; 3.1336x over previous
import functools

import jax
import jax.numpy as jnp
from jax import lax
from jax.experimental import pallas as pl
from jax.experimental.pallas import tpu as pltpu
from jax.experimental.pallas import tpu_sc as plsc

N = 10000
E = 320000
D = 128
H = 8
F = 16
HF = H * F
HH = HF // 2

NW = 32
EPT = E // NW
K = 80
NCH = EPT // K
PAIRS = (NCH - 1) // 2
STRIPE = 624
ZCH = 208
TAIL = N - 16 * STRIPE
BN = 400


def _dense_body(x_ref, wpt0_ref, wpt1_ref, wst_ref, asrc_ref, atrg_ref,
                ra_ref, proj0_ref, proj1_ref, skip_ref, srctab_ref,
                trgtab_ref):
    xb = x_ref[...]
    p0 = jnp.dot(xb, wpt0_ref[...], preferred_element_type=jnp.float32)
    p1 = jnp.dot(xb, wpt1_ref[...], preferred_element_type=jnp.float32)
    proj0_ref[...] = p0
    proj1_ref[...] = p1
    skip_ref[...] = jnp.dot(xb, wst_ref[...], preferred_element_type=jnp.float32)
    asrc = asrc_ref[...]
    atrg = atrg_ref[...]
    ra = ra_ref[...]
    srctab_ref[...] = (
        jnp.dot(p0 * asrc[:, :HH], ra[:HH], preferred_element_type=jnp.float32)
        + jnp.dot(p1 * asrc[:, HH:], ra[HH:], preferred_element_type=jnp.float32))
    trgtab_ref[...] = (
        jnp.dot(p0 * atrg[:, :HH], ra[:HH], preferred_element_type=jnp.float32)
        + jnp.dot(p1 * atrg[:, HH:], ra[HH:], preferred_element_type=jnp.float32))


def _dense_stage(x, wpt0, wpt1, wst, asrc, atrg, ra):
    grid = (N // BN,)
    return pl.pallas_call(
        _dense_body,
        grid=grid,
        in_specs=[
            pl.BlockSpec((BN, D), lambda i: (i, 0)),
            pl.BlockSpec((D, HH), lambda i: (0, 0)),
            pl.BlockSpec((D, HH), lambda i: (0, 0)),
            pl.BlockSpec((D, HF), lambda i: (0, 0)),
            pl.BlockSpec((1, HF), lambda i: (0, 0)),
            pl.BlockSpec((1, HF), lambda i: (0, 0)),
            pl.BlockSpec((HF, 16), lambda i: (0, 0)),
        ],
        out_specs=[
            pl.BlockSpec((BN, HH), lambda i: (i, 0)),
            pl.BlockSpec((BN, HH), lambda i: (i, 0)),
            pl.BlockSpec((BN, HF), lambda i: (i, 0)),
            pl.BlockSpec((BN, 16), lambda i: (i, 0)),
            pl.BlockSpec((BN, 16), lambda i: (i, 0)),
        ],
        out_shape=[
            jax.ShapeDtypeStruct((N, HH), jnp.float32),
            jax.ShapeDtypeStruct((N, HH), jnp.float32),
            jax.ShapeDtypeStruct((N, HF), jnp.float32),
            jax.ShapeDtypeStruct((N, 16), jnp.float32),
            jax.ShapeDtypeStruct((N, 16), jnp.float32),
        ],
    )(x, wpt0, wpt1, wst, asrc, atrg, ra)


def _sc_edge_body(srctab_hbm, trgtab_hbm, proj0_hbm, proj1_hbm, src_hbm,
                  trg3_hbm, prob_hbm, c_hbm,
                  denom_hbm, numa_hbm, numb_hbm,
                  src_v, trg2_v, prob_v, a_v, b_v, p_v, e_v, w_v, c_v,
                  z64_v, z16_v, denom_sh, numer_sh,
                  gsem0, gsem1, ssem0, ssem1):
    cid = lax.axis_index("c")
    sid = lax.axis_index("s")
    wid = sid * 2 + cid
    base = wid * EPT

    pltpu.sync_copy(src_hbm.at[pl.ds(base, EPT)], src_v)
    pltpu.sync_copy(trg3_hbm.at[wid], trg2_v)
    pltpu.sync_copy(prob_hbm.at[pl.ds(base, EPT)], prob_v.at[pl.ds(0, EPT)])
    pltpu.sync_copy(c_hbm, c_v)
    c_vec = c_v[...]

    zv = jnp.zeros((16,), jnp.float32)

    def zfill(r, _):
        for h in range(4):
            z64_v[r, pl.ds(h * 16, 16)] = zv
        z16_v[r, :] = zv
        return 0

    lax.fori_loop(0, ZCH, zfill, 0)

    def zero_stripes(zero_denom):
        for kk in range(STRIPE // ZCH):
            r0 = sid * STRIPE + kk * ZCH
            pltpu.sync_copy(z64_v, numer_sh.at[pl.ds(r0, ZCH)])
            if zero_denom:
                pltpu.sync_copy(z16_v, denom_sh.at[pl.ds(r0, ZCH)])

        @pl.when(sid == 15)
        def _zero_tail():
            pltpu.sync_copy(z64_v.at[pl.ds(0, TAIL)],
                            numer_sh.at[pl.ds(16 * STRIPE, TAIL)])
            if zero_denom:
                pltpu.sync_copy(z16_v.at[pl.ds(0, TAIL)],
                                denom_sh.at[pl.ds(16 * STRIPE, TAIL)])

    zero_stripes(True)
    plsc.subcore_barrier()

    def run_pass(proj_hbm, h0, scatter_denom):
        gsems = (gsem0, gsem1)
        ssems = (ssem0, ssem1)

        def fire(j, par):
            src_slice = src_v.at[pl.ds(j * K, K)]
            po = par * K
            pltpu.async_copy(srctab_hbm.at[src_slice],
                             a_v.at[pl.ds(po, K)], gsems[par])
            pltpu.async_copy(trgtab_hbm.at[trg2_v.at[j]],
                             b_v.at[pl.ds(po, K)], gsems[par])
            pltpu.async_copy(proj_hbm.at[src_slice],
                             p_v.at[pl.ds(po, K)], gsems[par])

        def wait_gathers(j, par):
            src_slice = src_v.at[pl.ds(j * K, K)]
            po = par * K
            pltpu.make_async_copy(srctab_hbm.at[src_slice],
                                  a_v.at[pl.ds(po, K)], gsems[par]).wait()
            pltpu.make_async_copy(trgtab_hbm.at[src_slice],
                                  b_v.at[pl.ds(po, K)], gsems[par]).wait()
            pltpu.make_async_copy(proj_hbm.at[src_slice],
                                  p_v.at[pl.ds(po, K)], gsems[par]).wait()

        def wait_scatters(j, par):
            po = par * K
            trg_row = trg2_v.at[j]
            if scatter_denom:
                pltpu.make_async_copy(e_v.at[pl.ds(po, K)],
                                      denom_sh.at[trg_row], ssems[par]).wait()
            pltpu.make_async_copy(w_v.at[pl.ds(po, K)],
                                  numer_sh.at[trg_row], ssems[par]).wait()

        def compute_scatter(j, par):
            po = par * K

            @plsc.parallel_loop(0, K, unroll=4)
            def edge(i):
                a = a_v[po + i, :]
                bb = b_v[po + i, :]
                pv = prob_v[pl.ds(j * K + i, 16)]
                s = a + bb + pv[0] * c_vec
                s = jnp.where(s > 0.0, s, 0.2 * s)
                e = jnp.exp(s)
                if scatter_denom:
                    e_v[po + i, :] = e
                for h in range(4):
                    w_v[po + i, pl.ds(h * 16, 16)] = (
                        p_v[po + i, pl.ds(h * 16, 16)] * e[h0 + h])

            trg_row = trg2_v.at[j]
            if scatter_denom:
                pltpu.async_copy(e_v.at[pl.ds(po, K)],
                                 denom_sh.at[trg_row], ssems[par], add=True)
            pltpu.async_copy(w_v.at[pl.ds(po, K)],
                             numer_sh.at[trg_row], ssems[par], add=True)

        fire(0, 0)

        def pair(jj, _):
            for b in range(2):
                j = jj * 2 + b
                if b == 0:
                    fire(j + 1, 1)
                else:
                    @pl.when(jj < PAIRS - 1)
                    def _fire_next():
                        fire(j + 1, 0)

                wait_gathers(j, b)

                @pl.when(jj >= 1)
                def _reuse_wait():
                    wait_scatters(j, b)

                compute_scatter(j, b)
            return 0

        lax.fori_loop(0, PAIRS, pair, 0)
        fire(NCH - 1, 0)
        wait_scatters(NCH - 1, 0)
        wait_gathers(NCH - 1, 0)
        compute_scatter(NCH - 1, 0)
        wait_scatters(0, 1)
        wait_scatters(0, 0)

    def writeback(src_sh, dst_hbm, width_tail_buf):
        out_base = cid * N + sid * STRIPE
        pltpu.sync_copy(src_sh.at[pl.ds(sid * STRIPE, STRIPE)],
                        dst_hbm.at[pl.ds(out_base, STRIPE)])

        @pl.when(sid == 15)
        def _tail():
            pltpu.sync_copy(src_sh.at[pl.ds(16 * STRIPE, TAIL)],
                            dst_hbm.at[pl.ds(cid * N + 16 * STRIPE, TAIL)])

    run_pass(proj0_hbm, 0, True)
    plsc.subcore_barrier()
    writeback(denom_sh, denom_hbm, z16_v)
    writeback(numer_sh, numa_hbm, z64_v)
    zero_stripes(False)
    plsc.subcore_barrier()

    run_pass(proj1_hbm, 4, False)
    plsc.subcore_barrier()
    writeback(numer_sh, numb_hbm, z64_v)


def _sc_edge_stage(srctab, trgtab, proj0, proj1, src, trg3, prob, cvec):
    mesh = plsc.VectorSubcoreMesh(core_axis_name="c", subcore_axis_name="s")
    fn = pl.kernel(
        _sc_edge_body,
        compiler_params=pltpu.CompilerParams(use_tc_tiling_on_sc=False),
        out_type=[
            jax.ShapeDtypeStruct((2 * N, 16), jnp.float32),
            jax.ShapeDtypeStruct((2 * N, HH), jnp.float32),
            jax.ShapeDtypeStruct((2 * N, HH), jnp.float32),
        ],
        mesh=mesh,
        scratch_types=[
            pltpu.VMEM((EPT,), jnp.int32),
            pltpu.VMEM((NCH, K), jnp.int32),
            pltpu.VMEM((EPT + 16,), jnp.float32),
            pltpu.VMEM((2 * K, 16), jnp.float32),
            pltpu.VMEM((2 * K, 16), jnp.float32),
            pltpu.VMEM((2 * K, HH), jnp.float32),
            pltpu.VMEM((2 * K, 16), jnp.float32),
            pltpu.VMEM((2 * K, HH), jnp.float32),
            pltpu.VMEM((16,), jnp.float32),
            pltpu.VMEM((ZCH, HH), jnp.float32),
            pltpu.VMEM((ZCH, 16), jnp.float32),
            pltpu.VMEM_SHARED((N, 16), jnp.float32),
            pltpu.VMEM_SHARED((N, HH), jnp.float32),
            pltpu.SemaphoreType.DMA,
            pltpu.SemaphoreType.DMA,
            pltpu.SemaphoreType.DMA,
            pltpu.SemaphoreType.DMA,
        ],
    )
    return fn(srctab, trgtab, proj0, proj1, src, trg3, prob, cvec)


def _combine_body(na0_ref, na1_ref, nb0_ref, nb1_ref, d0_ref, d1_ref,
                  skip_ref, bias_ref, r_ref, out_ref):
    d = jnp.dot(d0_ref[...] + d1_ref[...], r_ref[...],
                preferred_element_type=jnp.float32) + 1e-16
    num = jnp.concatenate([na0_ref[...] + na1_ref[...],
                           nb0_ref[...] + nb1_ref[...]], axis=-1)
    y = num / d + skip_ref[...] + bias_ref[...]
    out_ref[...] = jnp.where(y > 0.0, y, jnp.exp(jnp.minimum(y, 0.0)) - 1.0)


def _combine_stage(na0, na1, nb0, nb1, d0, d1, skip, bias, r):
    grid = (N // BN,)
    return pl.pallas_call(
        _combine_body,
        grid=grid,
        in_specs=[
            pl.BlockSpec((BN, HH), lambda i: (i, 0)),
            pl.BlockSpec((BN, HH), lambda i: (i, 0)),
            pl.BlockSpec((BN, HH), lambda i: (i, 0)),
            pl.BlockSpec((BN, HH), lambda i: (i, 0)),
            pl.BlockSpec((BN, 16), lambda i: (i, 0)),
            pl.BlockSpec((BN, 16), lambda i: (i, 0)),
            pl.BlockSpec((BN, HF), lambda i: (i, 0)),
            pl.BlockSpec((1, HF), lambda i: (0, 0)),
            pl.BlockSpec((16, HF), lambda i: (0, 0)),
        ],
        out_specs=pl.BlockSpec((BN, HF), lambda i: (i, 0)),
        out_shape=jax.ShapeDtypeStruct((N, HF), jnp.float32),
    )(na0, na1, nb0, nb1, d0, d1, skip, bias, r)


def kernel(x, edge_index, edge_prob, W_proj, W_tp, a_src, a_trg, a_tp,
           W_skip, bias):
    wpt = W_proj.T
    wst = W_skip.T
    asrc = a_src.reshape(1, HF)
    atrg = a_trg.reshape(1, HF)
    hf_i = lax.broadcasted_iota(jnp.int32, (HF, 16), 0)
    col_i = lax.broadcasted_iota(jnp.int32, (HF, 16), 1)
    ra = (hf_i // F == col_i).astype(jnp.float32)
    row_i = lax.broadcasted_iota(jnp.int32, (16, HF), 0)
    hf_j = lax.broadcasted_iota(jnp.int32, (16, HF), 1)
    r = (row_i == hf_j // F).astype(jnp.float32)
    c8 = (W_tp[:, 0] * a_tp.reshape(HF)).reshape(H, F).sum(-1)
    c16 = jnp.concatenate([c8, c8]).astype(jnp.float32)

    proj0, proj1, skip, srctab, trgtab = _dense_stage(
        x, wpt[:, :HH], wpt[:, HH:], wst, asrc, atrg, ra)

    src = edge_index[0]
    trg3 = edge_index[1].reshape(NW, NCH, K)
    prob = edge_prob[:, 0]
    denom, numa, numb = _sc_edge_stage(srctab, trgtab, proj0, proj1, src,
                                       trg3, prob, c16)

    out = _combine_stage(numa[:N], numa[N:], numb[:N], numb[N:],
                         denom[:N], denom[N:], skip, bias.reshape(1, HF), r)
    return (out, edge_index, edge_prob)


# speedup vs baseline: 120.4646x; 1.1063x over previous
import functools

import numpy as np
import jax
import jax.numpy as jnp
from jax import lax
from jax.experimental import pallas as pl
from jax.experimental.pallas import tpu as pltpu
from jax.experimental.pallas import tpu_sc as plsc

N = 10000
E = 320000
D = 128
H = 8
F = 16
HF = H * F
HH = HF // 2

NW = 32
EPT = E // NW
K = 80
NCH = EPT // K
PAIRS = (NCH - 1) // 2
STRIPE = 624
ZCH = 208
TAIL = N - 16 * STRIPE
BN = 400

_RA = np.equal.outer(np.arange(HF) // F, np.arange(16)).astype(np.float32)
_RB = np.equal.outer(np.arange(16), np.arange(HF) // F).astype(np.float32)


def _dense_body(x_ref, wp0_ref, wp1_ref, ws_ref, asrc_ref, atrg_ref,
                wtp_ref, atp_ref, ra_ref, proj0_ref, proj1_ref, skip_ref,
                srctab_ref, trgtab_ref, c_ref):
    xb = x_ref[...]
    dn = (((1,), (1,)), ((), ()))
    p0 = lax.dot_general(xb, wp0_ref[...], dn,
                         preferred_element_type=jnp.float32)
    p1 = lax.dot_general(xb, wp1_ref[...], dn,
                         preferred_element_type=jnp.float32)
    proj0_ref[...] = p0
    proj1_ref[...] = p1
    skip_ref[...] = lax.dot_general(xb, ws_ref[...], dn,
                                    preferred_element_type=jnp.float32)
    asrc = asrc_ref[...]
    atrg = atrg_ref[...]
    ra = ra_ref[...]
    srctab_ref[...] = (
        jnp.dot(p0 * asrc[:, :HH], ra[:HH], preferred_element_type=jnp.float32)
        + jnp.dot(p1 * asrc[:, HH:], ra[HH:], preferred_element_type=jnp.float32))
    trgtab_ref[...] = (
        jnp.dot(p0 * atrg[:, :HH], ra[:HH], preferred_element_type=jnp.float32)
        + jnp.dot(p1 * atrg[:, HH:], ra[HH:], preferred_element_type=jnp.float32))
    c_ref[...] = jnp.dot(wtp_ref[...] * atp_ref[...], ra,
                         preferred_element_type=jnp.float32)


def _dense_stage(x, W_proj, W_skip, asrc, atrg, wtp, atp, ra):
    grid = (N // BN,)
    return pl.pallas_call(
        _dense_body,
        grid=grid,
        in_specs=[
            pl.BlockSpec((BN, D), lambda i: (i, 0)),
            pl.BlockSpec((HH, D), lambda i: (0, 0)),
            pl.BlockSpec((HH, D), lambda i: (1, 0)),
            pl.BlockSpec((HF, D), lambda i: (0, 0)),
            pl.BlockSpec((1, HF), lambda i: (0, 0)),
            pl.BlockSpec((1, HF), lambda i: (0, 0)),
            pl.BlockSpec((1, HF), lambda i: (0, 0)),
            pl.BlockSpec((1, HF), lambda i: (0, 0)),
            pl.BlockSpec((HF, 16), lambda i: (0, 0)),
        ],
        out_specs=[
            pl.BlockSpec((BN, HH), lambda i: (i, 0)),
            pl.BlockSpec((BN, HH), lambda i: (i, 0)),
            pl.BlockSpec((BN, HF), lambda i: (i, 0)),
            pl.BlockSpec((BN, 16), lambda i: (i, 0)),
            pl.BlockSpec((BN, 16), lambda i: (i, 0)),
            pl.BlockSpec((1, 16), lambda i: (0, 0)),
        ],
        out_shape=[
            jax.ShapeDtypeStruct((N, HH), jnp.float32),
            jax.ShapeDtypeStruct((N, HH), jnp.float32),
            jax.ShapeDtypeStruct((N, HF), jnp.float32),
            jax.ShapeDtypeStruct((N, 16), jnp.float32),
            jax.ShapeDtypeStruct((N, 16), jnp.float32),
            jax.ShapeDtypeStruct((1, 16), jnp.float32),
        ],
    )(x, W_proj, W_proj, W_skip, asrc, atrg, wtp, atp, ra)


def _sc_edge_body(srctab_hbm, trgtab_hbm, proj0_hbm, proj1_hbm, ei_hbm,
                  prob_hbm, c_hbm,
                  denom_hbm, numa_hbm, numb_hbm,
                  src_v, trg_v, prob_v, a_v, b_v, p_v, e_v, w_v, c_v,
                  z64_v, z16_v, denom_sh, numer_sh,
                  gsem0, gsem1, ssem0, ssem1):
    cid = lax.axis_index("c")
    sid = lax.axis_index("s")
    wid = sid * 2 + cid
    base = wid * EPT

    pltpu.sync_copy(ei_hbm.at[0, pl.ds(base, EPT)], src_v)
    pltpu.sync_copy(ei_hbm.at[1, pl.ds(base, EPT)], trg_v)
    pltpu.sync_copy(prob_hbm.at[pl.ds(base, EPT)], prob_v.at[pl.ds(0, EPT)])
    pltpu.sync_copy(c_hbm, c_v)
    c_vec = c_v[0, :]

    zv = jnp.zeros((16,), jnp.float32)

    def zfill(r, _):
        for h in range(4):
            z64_v[r, pl.ds(h * 16, 16)] = zv
        z16_v[r, :] = zv
        return 0

    lax.fori_loop(0, ZCH, zfill, 0)

    def zero_stripes(zero_denom):
        for kk in range(STRIPE // ZCH):
            r0 = sid * STRIPE + kk * ZCH
            pltpu.sync_copy(z64_v, numer_sh.at[pl.ds(r0, ZCH)])
            if zero_denom:
                pltpu.sync_copy(z16_v, denom_sh.at[pl.ds(r0, ZCH)])

        @pl.when(sid == 15)
        def _zero_tail():
            pltpu.sync_copy(z64_v.at[pl.ds(0, TAIL)],
                            numer_sh.at[pl.ds(16 * STRIPE, TAIL)])
            if zero_denom:
                pltpu.sync_copy(z16_v.at[pl.ds(0, TAIL)],
                                denom_sh.at[pl.ds(16 * STRIPE, TAIL)])

    zero_stripes(True)
    plsc.subcore_barrier()

    def run_pass(proj_hbm, h0, scatter_denom):
        gsems = (gsem0, gsem1)
        ssems = (ssem0, ssem1)

        def fire(j, par):
            src_slice = src_v.at[pl.ds(j * K, K)]
            po = par * K
            pltpu.async_copy(srctab_hbm.at[src_slice],
                             a_v.at[pl.ds(po, K)], gsems[par])
            pltpu.async_copy(trgtab_hbm.at[trg_v.at[pl.ds(j * K, K)]],
                             b_v.at[pl.ds(po, K)], gsems[par])
            pltpu.async_copy(proj_hbm.at[src_slice],
                             p_v.at[pl.ds(po, K)], gsems[par])

        def wait_gathers(j, par):
            src_slice = src_v.at[pl.ds(j * K, K)]
            po = par * K
            pltpu.make_async_copy(srctab_hbm.at[src_slice],
                                  a_v.at[pl.ds(po, K)], gsems[par]).wait()
            pltpu.make_async_copy(trgtab_hbm.at[src_slice],
                                  b_v.at[pl.ds(po, K)], gsems[par]).wait()
            pltpu.make_async_copy(proj_hbm.at[src_slice],
                                  p_v.at[pl.ds(po, K)], gsems[par]).wait()

        def wait_scatters(j, par):
            po = par * K
            trg_row = trg_v.at[pl.ds(j * K, K)]
            if scatter_denom:
                pltpu.make_async_copy(e_v.at[pl.ds(po, K)],
                                      denom_sh.at[trg_row], ssems[par]).wait()
            pltpu.make_async_copy(w_v.at[pl.ds(po, K)],
                                  numer_sh.at[trg_row], ssems[par]).wait()

        def compute_scatter(j, par):
            po = par * K

            @plsc.parallel_loop(0, K, unroll=4)
            def edge(i):
                a = a_v[po + i, :]
                bb = b_v[po + i, :]
                pv = prob_v[pl.ds(j * K + i, 16)]
                s = a + bb + pv[0] * c_vec
                s = jnp.where(s > 0.0, s, 0.2 * s)
                e = jnp.exp(s)
                if scatter_denom:
                    e_v[po + i, :] = e
                for h in range(4):
                    w_v[po + i, pl.ds(h * 16, 16)] = (
                        p_v[po + i, pl.ds(h * 16, 16)] * e[h0 + h])

            trg_row = trg_v.at[pl.ds(j * K, K)]
            if scatter_denom:
                pltpu.async_copy(e_v.at[pl.ds(po, K)],
                                 denom_sh.at[trg_row], ssems[par], add=True)
            pltpu.async_copy(w_v.at[pl.ds(po, K)],
                             numer_sh.at[trg_row], ssems[par], add=True)

        fire(0, 0)

        def pair(jj, _):
            for b in range(2):
                j = jj * 2 + b
                if b == 0:
                    fire(j + 1, 1)
                else:
                    @pl.when(jj < PAIRS - 1)
                    def _fire_next():
                        fire(j + 1, 0)

                wait_gathers(j, b)

                @pl.when(jj >= 1)
                def _reuse_wait():
                    wait_scatters(j, b)

                compute_scatter(j, b)
            return 0

        lax.fori_loop(0, PAIRS, pair, 0)
        fire(NCH - 1, 0)
        wait_scatters(NCH - 1, 0)
        wait_gathers(NCH - 1, 0)
        compute_scatter(NCH - 1, 0)
        wait_scatters(0, 1)
        wait_scatters(0, 0)

    def writeback(src_sh, dst_hbm):
        out_base = cid * N + sid * STRIPE
        pltpu.sync_copy(src_sh.at[pl.ds(sid * STRIPE, STRIPE)],
                        dst_hbm.at[pl.ds(out_base, STRIPE)])

        @pl.when(sid == 15)
        def _tail():
            pltpu.sync_copy(src_sh.at[pl.ds(16 * STRIPE, TAIL)],
                            dst_hbm.at[pl.ds(cid * N + 16 * STRIPE, TAIL)])

    run_pass(proj0_hbm, 0, True)
    plsc.subcore_barrier()
    writeback(denom_sh, denom_hbm)
    writeback(numer_sh, numa_hbm)
    zero_stripes(False)
    plsc.subcore_barrier()

    run_pass(proj1_hbm, 4, False)
    plsc.subcore_barrier()
    writeback(numer_sh, numb_hbm)


def _sc_edge_stage(srctab, trgtab, proj0, proj1, edge_index, prob, cvec):
    mesh = plsc.VectorSubcoreMesh(core_axis_name="c", subcore_axis_name="s")
    fn = pl.kernel(
        _sc_edge_body,
        compiler_params=pltpu.CompilerParams(use_tc_tiling_on_sc=False),
        out_type=[
            jax.ShapeDtypeStruct((2 * N, 16), jnp.float32),
            jax.ShapeDtypeStruct((2 * N, HH), jnp.float32),
            jax.ShapeDtypeStruct((2 * N, HH), jnp.float32),
        ],
        mesh=mesh,
        scratch_types=[
            pltpu.VMEM((EPT,), jnp.int32),
            pltpu.VMEM((EPT,), jnp.int32),
            pltpu.VMEM((EPT + 16,), jnp.float32),
            pltpu.VMEM((2 * K, 16), jnp.float32),
            pltpu.VMEM((2 * K, 16), jnp.float32),
            pltpu.VMEM((2 * K, HH), jnp.float32),
            pltpu.VMEM((2 * K, 16), jnp.float32),
            pltpu.VMEM((2 * K, HH), jnp.float32),
            pltpu.VMEM((1, 16), jnp.float32),
            pltpu.VMEM((ZCH, HH), jnp.float32),
            pltpu.VMEM((ZCH, 16), jnp.float32),
            pltpu.VMEM_SHARED((N, 16), jnp.float32),
            pltpu.VMEM_SHARED((N, HH), jnp.float32),
            pltpu.SemaphoreType.DMA,
            pltpu.SemaphoreType.DMA,
            pltpu.SemaphoreType.DMA,
            pltpu.SemaphoreType.DMA,
        ],
    )
    return fn(srctab, trgtab, proj0, proj1, edge_index, prob, cvec)


def _combine_body(na0_ref, na1_ref, nb0_ref, nb1_ref, d0_ref, d1_ref,
                  skip_ref, bias_ref, r_ref, out_ref):
    d = jnp.dot(d0_ref[...] + d1_ref[...], r_ref[...],
                preferred_element_type=jnp.float32) + 1e-16
    num = jnp.concatenate([na0_ref[...] + na1_ref[...],
                           nb0_ref[...] + nb1_ref[...]], axis=-1)
    y = num / d + skip_ref[...] + bias_ref[...]
    out_ref[...] = jnp.where(y > 0.0, y, jnp.exp(jnp.minimum(y, 0.0)) - 1.0)


def _combine_stage(numa, numb, denom, skip, bias, r):
    grid = (N // BN,)
    half = N // BN
    return pl.pallas_call(
        _combine_body,
        grid=grid,
        in_specs=[
            pl.BlockSpec((BN, HH), lambda i: (i, 0)),
            pl.BlockSpec((BN, HH), lambda i: (i + half, 0)),
            pl.BlockSpec((BN, HH), lambda i: (i, 0)),
            pl.BlockSpec((BN, HH), lambda i: (i + half, 0)),
            pl.BlockSpec((BN, 16), lambda i: (i, 0)),
            pl.BlockSpec((BN, 16), lambda i: (i + half, 0)),
            pl.BlockSpec((BN, HF), lambda i: (i, 0)),
            pl.BlockSpec((1, HF), lambda i: (0, 0)),
            pl.BlockSpec((16, HF), lambda i: (0, 0)),
        ],
        out_specs=pl.BlockSpec((BN, HF), lambda i: (i, 0)),
        out_shape=jax.ShapeDtypeStruct((N, HF), jnp.float32),
    )(numa, numa, numb, numb, denom, denom, skip, bias, r)


def kernel(x, edge_index, edge_prob, W_proj, W_tp, a_src, a_trg, a_tp,
           W_skip, bias):
    asrc = a_src.reshape(1, HF)
    atrg = a_trg.reshape(1, HF)
    wtp = W_tp.reshape(1, HF)
    atp = a_tp.reshape(1, HF)
    ra = jnp.asarray(_RA)
    r = jnp.asarray(_RB)

    proj0, proj1, skip, srctab, trgtab, c2 = _dense_stage(
        x, W_proj, W_skip, asrc, atrg, wtp, atp, ra)

    prob = edge_prob.reshape(E)
    denom, numa, numb = _sc_edge_stage(srctab, trgtab, proj0, proj1,
                                       edge_index, prob, c2)

    out = _combine_stage(numa, numb, denom, skip, bias.reshape(1, HF), r)
    return (out, edge_index, edge_prob)


# speedup vs baseline: 123.2014x; 1.0227x over previous
import functools

import numpy as np
import jax
import jax.numpy as jnp
from jax import lax
from jax.experimental import pallas as pl
from jax.experimental.pallas import tpu as pltpu
from jax.experimental.pallas import tpu_sc as plsc

N = 10000
E = 320000
D = 128
H = 8
F = 16
HF = H * F
HH = HF // 2
PW = HH + 16

NW = 32
EPT = E // NW
K = 80
NCH = EPT // K
PAIRS = (NCH - 1) // 2
STRIPE = 624
ZCH = 208
TAIL = N - 16 * STRIPE
BN = 400

_RA = np.equal.outer(np.arange(HF) // F, np.arange(16)).astype(np.float32)
_RB = np.equal.outer(np.arange(16), np.arange(HF) // F).astype(np.float32)


def _dense_body(x_ref, wp0_ref, wp1_ref, ws_ref, asrc_ref, atrg_ref,
                wtp_ref, atp_ref, ra_ref, ptab0_ref, ptab1_ref, skip_ref,
                trgtab_ref, c_ref):
    xb = x_ref[...]
    dn = (((1,), (1,)), ((), ()))
    p0 = lax.dot_general(xb, wp0_ref[...], dn,
                         preferred_element_type=jnp.float32)
    p1 = lax.dot_general(xb, wp1_ref[...], dn,
                         preferred_element_type=jnp.float32)
    skip_ref[...] = lax.dot_general(xb, ws_ref[...], dn,
                                    preferred_element_type=jnp.float32)
    asrc = asrc_ref[...]
    atrg = atrg_ref[...]
    ra = ra_ref[...]
    stab = (
        jnp.dot(p0 * asrc[:, :HH], ra[:HH], preferred_element_type=jnp.float32)
        + jnp.dot(p1 * asrc[:, HH:], ra[HH:], preferred_element_type=jnp.float32))
    ptab0_ref[...] = jnp.concatenate([p0, stab], axis=-1)
    ptab1_ref[...] = jnp.concatenate([p1, stab], axis=-1)
    trgtab_ref[...] = (
        jnp.dot(p0 * atrg[:, :HH], ra[:HH], preferred_element_type=jnp.float32)
        + jnp.dot(p1 * atrg[:, HH:], ra[HH:], preferred_element_type=jnp.float32))
    c_ref[...] = jnp.dot(wtp_ref[...] * atp_ref[...], ra,
                         preferred_element_type=jnp.float32)


def _dense_stage(x, W_proj, W_skip, asrc, atrg, wtp, atp, ra):
    grid = (N // BN,)
    return pl.pallas_call(
        _dense_body,
        grid=grid,
        in_specs=[
            pl.BlockSpec((BN, D), lambda i: (i, 0)),
            pl.BlockSpec((HH, D), lambda i: (0, 0)),
            pl.BlockSpec((HH, D), lambda i: (1, 0)),
            pl.BlockSpec((HF, D), lambda i: (0, 0)),
            pl.BlockSpec((1, HF), lambda i: (0, 0)),
            pl.BlockSpec((1, HF), lambda i: (0, 0)),
            pl.BlockSpec((1, HF), lambda i: (0, 0)),
            pl.BlockSpec((1, HF), lambda i: (0, 0)),
            pl.BlockSpec((HF, 16), lambda i: (0, 0)),
        ],
        out_specs=[
            pl.BlockSpec((BN, PW), lambda i: (i, 0)),
            pl.BlockSpec((BN, PW), lambda i: (i, 0)),
            pl.BlockSpec((BN, HF), lambda i: (i, 0)),
            pl.BlockSpec((BN, 16), lambda i: (i, 0)),
            pl.BlockSpec((1, 16), lambda i: (0, 0)),
        ],
        out_shape=[
            jax.ShapeDtypeStruct((N, PW), jnp.float32),
            jax.ShapeDtypeStruct((N, PW), jnp.float32),
            jax.ShapeDtypeStruct((N, HF), jnp.float32),
            jax.ShapeDtypeStruct((N, 16), jnp.float32),
            jax.ShapeDtypeStruct((1, 16), jnp.float32),
        ],
    )(x, W_proj, W_proj, W_skip, asrc, atrg, wtp, atp, ra)


def _sc_edge_body(ptab0_hbm, ptab1_hbm, trgtab_hbm, ei_hbm, prob_hbm, c_hbm,
                  numa_hbm, numb_hbm,
                  src_v, trg_v, prob_v, b_v, p_v, w_v, c_v, z80_v,
                  acc_sh, gsem0, gsem1, ssem0, ssem1):
    cid = lax.axis_index("c")
    sid = lax.axis_index("s")
    wid = sid * 2 + cid
    base = wid * EPT

    pltpu.sync_copy(ei_hbm.at[0, pl.ds(base, EPT)], src_v)
    pltpu.sync_copy(ei_hbm.at[1, pl.ds(base, EPT)], trg_v)
    pltpu.sync_copy(prob_hbm.at[pl.ds(base, EPT)], prob_v.at[pl.ds(0, EPT)])
    pltpu.sync_copy(c_hbm, c_v)
    c_vec = c_v[0, :]

    zv = jnp.zeros((16,), jnp.float32)

    def zfill(r, _):
        for h in range(5):
            z80_v[r, pl.ds(h * 16, 16)] = zv
        return 0

    lax.fori_loop(0, ZCH, zfill, 0)

    def zero_stripes():
        for kk in range(STRIPE // ZCH):
            r0 = sid * STRIPE + kk * ZCH
            pltpu.sync_copy(z80_v, acc_sh.at[pl.ds(r0, ZCH)])

        @pl.when(sid == 15)
        def _zero_tail():
            pltpu.sync_copy(z80_v.at[pl.ds(0, TAIL)],
                            acc_sh.at[pl.ds(16 * STRIPE, TAIL)])

    zero_stripes()
    plsc.subcore_barrier()

    def run_pass(ptab_hbm, h0):
        gsems = (gsem0, gsem1)
        ssems = (ssem0, ssem1)

        def fire(j, par):
            po = par * K
            pltpu.async_copy(ptab_hbm.at[src_v.at[pl.ds(j * K, K)]],
                             p_v.at[pl.ds(po, K)], gsems[par])
            pltpu.async_copy(trgtab_hbm.at[trg_v.at[pl.ds(j * K, K)]],
                             b_v.at[pl.ds(po, K)], gsems[par])

        def wait_gathers(j, par):
            po = par * K
            pltpu.make_async_copy(ptab_hbm.at[src_v.at[pl.ds(j * K, K)]],
                                  p_v.at[pl.ds(po, K)], gsems[par]).wait()
            pltpu.make_async_copy(trgtab_hbm.at[src_v.at[pl.ds(j * K, K)]],
                                  b_v.at[pl.ds(po, K)], gsems[par]).wait()

        def wait_scatters(j, par):
            po = par * K
            pltpu.make_async_copy(
                w_v.at[pl.ds(po, K)],
                acc_sh.at[trg_v.at[pl.ds(j * K, K)]], ssems[par]).wait()

        def compute_scatter(j, par):
            po = par * K

            @plsc.parallel_loop(0, K, unroll=4)
            def edge(i):
                a = p_v[po + i, pl.ds(HH, 16)]
                bb = b_v[po + i, :]
                pv = prob_v[pl.ds(j * K + i, 16)]
                s = a + bb + pv[0] * c_vec
                s = jnp.where(s > 0.0, s, 0.2 * s)
                e = jnp.exp(s)
                w_v[po + i, pl.ds(HH, 16)] = e
                for h in range(4):
                    w_v[po + i, pl.ds(h * 16, 16)] = (
                        p_v[po + i, pl.ds(h * 16, 16)] * e[h0 + h])

            pltpu.async_copy(w_v.at[pl.ds(po, K)],
                             acc_sh.at[trg_v.at[pl.ds(j * K, K)]],
                             ssems[par], add=True)

        fire(0, 0)

        def pair(jj, _):
            for b in range(2):
                j = jj * 2 + b
                if b == 0:
                    fire(j + 1, 1)
                else:
                    @pl.when(jj < PAIRS - 1)
                    def _fire_next():
                        fire(j + 1, 0)

                wait_gathers(j, b)

                @pl.when(jj >= 1)
                def _reuse_wait():
                    wait_scatters(j, b)

                compute_scatter(j, b)
            return 0

        lax.fori_loop(0, PAIRS, pair, 0)
        fire(NCH - 1, 0)
        wait_scatters(NCH - 1, 0)
        wait_gathers(NCH - 1, 0)
        compute_scatter(NCH - 1, 0)
        wait_scatters(0, 1)
        wait_scatters(0, 0)

    def writeback(dst_hbm):
        out_base = cid * N + sid * STRIPE
        pltpu.sync_copy(acc_sh.at[pl.ds(sid * STRIPE, STRIPE)],
                        dst_hbm.at[pl.ds(out_base, STRIPE)])

        @pl.when(sid == 15)
        def _tail():
            pltpu.sync_copy(acc_sh.at[pl.ds(16 * STRIPE, TAIL)],
                            dst_hbm.at[pl.ds(cid * N + 16 * STRIPE, TAIL)])

    run_pass(ptab0_hbm, 0)
    plsc.subcore_barrier()
    writeback(numa_hbm)
    zero_stripes()
    plsc.subcore_barrier()

    run_pass(ptab1_hbm, 4)
    plsc.subcore_barrier()
    writeback(numb_hbm)


def _sc_edge_stage(ptab0, ptab1, trgtab, edge_index, prob, cvec):
    mesh = plsc.VectorSubcoreMesh(core_axis_name="c", subcore_axis_name="s")
    fn = pl.kernel(
        _sc_edge_body,
        compiler_params=pltpu.CompilerParams(use_tc_tiling_on_sc=False),
        out_type=[
            jax.ShapeDtypeStruct((2 * N, PW), jnp.float32),
            jax.ShapeDtypeStruct((2 * N, PW), jnp.float32),
        ],
        mesh=mesh,
        scratch_types=[
            pltpu.VMEM((EPT,), jnp.int32),
            pltpu.VMEM((EPT,), jnp.int32),
            pltpu.VMEM((EPT + 16,), jnp.float32),
            pltpu.VMEM((2 * K, 16), jnp.float32),
            pltpu.VMEM((2 * K, PW), jnp.float32),
            pltpu.VMEM((2 * K, PW), jnp.float32),
            pltpu.VMEM((1, 16), jnp.float32),
            pltpu.VMEM((ZCH, PW), jnp.float32),
            pltpu.VMEM_SHARED((N, PW), jnp.float32),
            pltpu.SemaphoreType.DMA,
            pltpu.SemaphoreType.DMA,
            pltpu.SemaphoreType.DMA,
            pltpu.SemaphoreType.DMA,
        ],
    )
    return fn(ptab0, ptab1, trgtab, edge_index, prob, cvec)


def _combine_body(na0_ref, na1_ref, nb0_ref, nb1_ref,
                  skip_ref, bias_ref, r_ref, out_ref):
    na = na0_ref[...] + na1_ref[...]
    nb = nb0_ref[...] + nb1_ref[...]
    d = jnp.dot(na[:, HH:], r_ref[...],
                preferred_element_type=jnp.float32) + 1e-16
    num = jnp.concatenate([na[:, :HH], nb[:, :HH]], axis=-1)
    y = num / d + skip_ref[...] + bias_ref[...]
    out_ref[...] = jnp.where(y > 0.0, y, jnp.exp(jnp.minimum(y, 0.0)) - 1.0)


def _combine_stage(numa, numb, skip, bias, r):
    grid = (N // BN,)
    half = N // BN
    return pl.pallas_call(
        _combine_body,
        grid=grid,
        in_specs=[
            pl.BlockSpec((BN, PW), lambda i: (i, 0)),
            pl.BlockSpec((BN, PW), lambda i: (i + half, 0)),
            pl.BlockSpec((BN, PW), lambda i: (i, 0)),
            pl.BlockSpec((BN, PW), lambda i: (i + half, 0)),
            pl.BlockSpec((BN, HF), lambda i: (i, 0)),
            pl.BlockSpec((1, HF), lambda i: (0, 0)),
            pl.BlockSpec((16, HF), lambda i: (0, 0)),
        ],
        out_specs=pl.BlockSpec((BN, HF), lambda i: (i, 0)),
        out_shape=jax.ShapeDtypeStruct((N, HF), jnp.float32),
    )(numa, numa, numb, numb, skip, bias, r)


def kernel(x, edge_index, edge_prob, W_proj, W_tp, a_src, a_trg, a_tp,
           W_skip, bias):
    asrc = a_src.reshape(1, HF)
    atrg = a_trg.reshape(1, HF)
    wtp = W_tp.reshape(1, HF)
    atp = a_tp.reshape(1, HF)
    ra = jnp.asarray(_RA)
    r = jnp.asarray(_RB)

    ptab0, ptab1, skip, trgtab, c2 = _dense_stage(
        x, W_proj, W_skip, asrc, atrg, wtp, atp, ra)

    prob = edge_prob.reshape(E)
    numa, numb = _sc_edge_stage(ptab0, ptab1, trgtab, edge_index, prob, c2)

    out = _combine_stage(numa, numb, skip, bias.reshape(1, HF), r)
    return (out, edge_index, edge_prob)


# speedup vs baseline: 123.6582x; 1.0037x over previous
import functools

import numpy as np
import jax
import jax.numpy as jnp
from jax import lax
from jax.experimental import pallas as pl
from jax.experimental.pallas import tpu as pltpu
from jax.experimental.pallas import tpu_sc as plsc

N = 10000
E = 320000
D = 128
H = 8
F = 16
HF = H * F
HH = HF // 2
PW = HH + 16

NW = 32
EPT = E // NW
K = 80
NCH = EPT // K
PAIRS = (NCH - 1) // 2
STRIPE = 624
ZCH = 208
TAIL = N - 16 * STRIPE
BN = 400

_RA = np.equal.outer(np.arange(HF) // F, np.arange(16)).astype(np.float32)
_RB = np.equal.outer(np.arange(16), np.arange(HF) // F).astype(np.float32)


def _dense_body(x_ref, wp0_ref, wp1_ref, ws_ref, asrc_ref, atrg_ref,
                wtp_ref, atp_ref, ra_ref, ptab0_ref, ptab1_ref, skip_ref,
                trgtab_ref, c_ref):
    xb = x_ref[...]
    dn = (((1,), (1,)), ((), ()))
    p0 = lax.dot_general(xb, wp0_ref[...], dn,
                         preferred_element_type=jnp.float32)
    p1 = lax.dot_general(xb, wp1_ref[...], dn,
                         preferred_element_type=jnp.float32)
    skip_ref[...] = lax.dot_general(xb, ws_ref[...], dn,
                                    preferred_element_type=jnp.float32)
    asrc = asrc_ref[...]
    atrg = atrg_ref[...]
    ra = ra_ref[...]
    stab = (
        jnp.dot(p0 * asrc[:, :HH], ra[:HH], preferred_element_type=jnp.float32)
        + jnp.dot(p1 * asrc[:, HH:], ra[HH:], preferred_element_type=jnp.float32))
    ptab0_ref[...] = jnp.concatenate([p0, stab], axis=-1)
    ptab1_ref[...] = jnp.concatenate([p1, stab], axis=-1)
    trgtab_ref[...] = (
        jnp.dot(p0 * atrg[:, :HH], ra[:HH], preferred_element_type=jnp.float32)
        + jnp.dot(p1 * atrg[:, HH:], ra[HH:], preferred_element_type=jnp.float32))
    c_ref[...] = jnp.dot(wtp_ref[...] * atp_ref[...], ra,
                         preferred_element_type=jnp.float32)


def _dense_stage(x, W_proj, W_skip, asrc, atrg, wtp, atp, ra):
    grid = (N // BN,)
    return pl.pallas_call(
        _dense_body,
        grid=grid,
        in_specs=[
            pl.BlockSpec((BN, D), lambda i: (i, 0)),
            pl.BlockSpec((HH, D), lambda i: (0, 0)),
            pl.BlockSpec((HH, D), lambda i: (1, 0)),
            pl.BlockSpec((HF, D), lambda i: (0, 0)),
            pl.BlockSpec((1, HF), lambda i: (0, 0)),
            pl.BlockSpec((1, HF), lambda i: (0, 0)),
            pl.BlockSpec((1, HF), lambda i: (0, 0)),
            pl.BlockSpec((1, HF), lambda i: (0, 0)),
            pl.BlockSpec((HF, 16), lambda i: (0, 0)),
        ],
        out_specs=[
            pl.BlockSpec((BN, PW), lambda i: (i, 0)),
            pl.BlockSpec((BN, PW), lambda i: (i, 0)),
            pl.BlockSpec((BN, HF), lambda i: (i, 0)),
            pl.BlockSpec((BN, 16), lambda i: (i, 0)),
            pl.BlockSpec((1, 16), lambda i: (0, 0)),
        ],
        out_shape=[
            jax.ShapeDtypeStruct((N, PW), jnp.float32),
            jax.ShapeDtypeStruct((N, PW), jnp.float32),
            jax.ShapeDtypeStruct((N, HF), jnp.float32),
            jax.ShapeDtypeStruct((N, 16), jnp.float32),
            jax.ShapeDtypeStruct((1, 16), jnp.float32),
        ],
    )(x, W_proj, W_proj, W_skip, asrc, atrg, wtp, atp, ra)


def _sc_edge_body(ptab0_hbm, ptab1_hbm, trgtab_hbm, ei_hbm, prob_hbm, c_hbm,
                  numa_hbm, numb_hbm,
                  src_v, trg_v, prob_v, b_v, p_v, w_v, c_v, z80_v,
                  acc_sh, gsem0, gsem1, ssem0, ssem1):
    cid = lax.axis_index("c")
    sid = lax.axis_index("s")
    wid = sid * 2 + cid
    base = wid * EPT

    pltpu.sync_copy(ei_hbm.at[0, pl.ds(base, EPT)], src_v)
    pltpu.sync_copy(ei_hbm.at[1, pl.ds(base, EPT)], trg_v)
    pltpu.sync_copy(prob_hbm.at[pl.ds(base, EPT)], prob_v.at[pl.ds(0, EPT)])
    pltpu.sync_copy(c_hbm, c_v)
    c_vec = c_v[0, :]

    zv = jnp.zeros((16,), jnp.float32)

    def zfill(r, _):
        for h in range(5):
            z80_v[r, pl.ds(h * 16, 16)] = zv
        return 0

    lax.fori_loop(0, ZCH, zfill, 0)

    def zero_stripes():
        for kk in range(STRIPE // ZCH):
            r0 = sid * STRIPE + kk * ZCH
            pltpu.sync_copy(z80_v, acc_sh.at[pl.ds(r0, ZCH)])

        @pl.when(sid == 15)
        def _zero_tail():
            pltpu.sync_copy(z80_v.at[pl.ds(0, TAIL)],
                            acc_sh.at[pl.ds(16 * STRIPE, TAIL)])

    zero_stripes()
    plsc.subcore_barrier()

    def run_pass(ptab_hbm, h0):
        gsems = (gsem0, gsem1)
        ssems = (ssem0, ssem1)

        def fire(j, par):
            po = par * K
            pltpu.async_copy(ptab_hbm.at[src_v.at[pl.ds(j * K, K)]],
                             p_v.at[pl.ds(po, K)], gsems[par])
            pltpu.async_copy(trgtab_hbm.at[trg_v.at[pl.ds(j * K, K)]],
                             b_v.at[pl.ds(po, K)], gsems[par])

        def wait_gathers(j, par):
            po = par * K
            pltpu.make_async_copy(ptab_hbm.at[src_v.at[pl.ds(j * K, K)]],
                                  p_v.at[pl.ds(po, K)], gsems[par]).wait()
            pltpu.make_async_copy(trgtab_hbm.at[src_v.at[pl.ds(j * K, K)]],
                                  b_v.at[pl.ds(po, K)], gsems[par]).wait()

        def wait_scatters(j, par):
            po = par * K
            pltpu.make_async_copy(
                w_v.at[pl.ds(po, K)],
                acc_sh.at[trg_v.at[pl.ds(j * K, K)]], ssems[par]).wait()

        def compute_scatter(j, par):
            po = par * K

            @plsc.parallel_loop(0, K, unroll=8)
            def edge(i):
                a = p_v[po + i, pl.ds(HH, 16)]
                bb = b_v[po + i, :]
                pv = prob_v[pl.ds(j * K + i, 16)]
                s = a + bb + pv[0] * c_vec
                s = jnp.where(s > 0.0, s, 0.2 * s)
                e = jnp.exp(s)
                w_v[po + i, pl.ds(HH, 16)] = e
                for h in range(4):
                    w_v[po + i, pl.ds(h * 16, 16)] = (
                        p_v[po + i, pl.ds(h * 16, 16)] * e[h0 + h])

            pltpu.async_copy(w_v.at[pl.ds(po, K)],
                             acc_sh.at[trg_v.at[pl.ds(j * K, K)]],
                             ssems[par], add=True)

        fire(0, 0)

        def pair(jj, _):
            for b in range(2):
                j = jj * 2 + b
                if b == 0:
                    fire(j + 1, 1)
                else:
                    @pl.when(jj < PAIRS - 1)
                    def _fire_next():
                        fire(j + 1, 0)

                wait_gathers(j, b)

                @pl.when(jj >= 1)
                def _reuse_wait():
                    wait_scatters(j, b)

                compute_scatter(j, b)
            return 0

        lax.fori_loop(0, PAIRS, pair, 0)
        fire(NCH - 1, 0)
        wait_scatters(NCH - 1, 0)
        wait_gathers(NCH - 1, 0)
        compute_scatter(NCH - 1, 0)
        wait_scatters(0, 1)
        wait_scatters(0, 0)

    def writeback(dst_hbm):
        out_base = cid * N + sid * STRIPE
        pltpu.sync_copy(acc_sh.at[pl.ds(sid * STRIPE, STRIPE)],
                        dst_hbm.at[pl.ds(out_base, STRIPE)])

        @pl.when(sid == 15)
        def _tail():
            pltpu.sync_copy(acc_sh.at[pl.ds(16 * STRIPE, TAIL)],
                            dst_hbm.at[pl.ds(cid * N + 16 * STRIPE, TAIL)])

    run_pass(ptab0_hbm, 0)
    plsc.subcore_barrier()
    writeback(numa_hbm)
    zero_stripes()
    plsc.subcore_barrier()

    run_pass(ptab1_hbm, 4)
    plsc.subcore_barrier()
    writeback(numb_hbm)


def _sc_edge_stage(ptab0, ptab1, trgtab, edge_index, prob, cvec):
    mesh = plsc.VectorSubcoreMesh(core_axis_name="c", subcore_axis_name="s")
    fn = pl.kernel(
        _sc_edge_body,
        compiler_params=pltpu.CompilerParams(use_tc_tiling_on_sc=False),
        out_type=[
            jax.ShapeDtypeStruct((2 * N, PW), jnp.float32),
            jax.ShapeDtypeStruct((2 * N, PW), jnp.float32),
        ],
        mesh=mesh,
        scratch_types=[
            pltpu.VMEM((EPT,), jnp.int32),
            pltpu.VMEM((EPT,), jnp.int32),
            pltpu.VMEM((EPT + 16,), jnp.float32),
            pltpu.VMEM((2 * K, 16), jnp.float32),
            pltpu.VMEM((2 * K, PW), jnp.float32),
            pltpu.VMEM((2 * K, PW), jnp.float32),
            pltpu.VMEM((1, 16), jnp.float32),
            pltpu.VMEM((ZCH, PW), jnp.float32),
            pltpu.VMEM_SHARED((N, PW), jnp.float32),
            pltpu.SemaphoreType.DMA,
            pltpu.SemaphoreType.DMA,
            pltpu.SemaphoreType.DMA,
            pltpu.SemaphoreType.DMA,
        ],
    )
    return fn(ptab0, ptab1, trgtab, edge_index, prob, cvec)


def _combine_body(na0_ref, na1_ref, nb0_ref, nb1_ref,
                  skip_ref, bias_ref, r_ref, out_ref):
    na = na0_ref[...] + na1_ref[...]
    nb = nb0_ref[...] + nb1_ref[...]
    d = jnp.dot(na[:, HH:], r_ref[...],
                preferred_element_type=jnp.float32) + 1e-16
    num = jnp.concatenate([na[:, :HH], nb[:, :HH]], axis=-1)
    y = num / d + skip_ref[...] + bias_ref[...]
    out_ref[...] = jnp.where(y > 0.0, y, jnp.exp(jnp.minimum(y, 0.0)) - 1.0)


def _combine_stage(numa, numb, skip, bias, r):
    grid = (N // BN,)
    half = N // BN
    return pl.pallas_call(
        _combine_body,
        grid=grid,
        in_specs=[
            pl.BlockSpec((BN, PW), lambda i: (i, 0)),
            pl.BlockSpec((BN, PW), lambda i: (i + half, 0)),
            pl.BlockSpec((BN, PW), lambda i: (i, 0)),
            pl.BlockSpec((BN, PW), lambda i: (i + half, 0)),
            pl.BlockSpec((BN, HF), lambda i: (i, 0)),
            pl.BlockSpec((1, HF), lambda i: (0, 0)),
            pl.BlockSpec((16, HF), lambda i: (0, 0)),
        ],
        out_specs=pl.BlockSpec((BN, HF), lambda i: (i, 0)),
        out_shape=jax.ShapeDtypeStruct((N, HF), jnp.float32),
    )(numa, numa, numb, numb, skip, bias, r)


def kernel(x, edge_index, edge_prob, W_proj, W_tp, a_src, a_trg, a_tp,
           W_skip, bias):
    asrc = a_src.reshape(1, HF)
    atrg = a_trg.reshape(1, HF)
    wtp = W_tp.reshape(1, HF)
    atp = a_tp.reshape(1, HF)
    ra = jnp.asarray(_RA)
    r = jnp.asarray(_RB)

    ptab0, ptab1, skip, trgtab, c2 = _dense_stage(
        x, W_proj, W_skip, asrc, atrg, wtp, atp, ra)

    prob = edge_prob.reshape(E)
    numa, numb = _sc_edge_stage(ptab0, ptab1, trgtab, edge_index, prob, c2)

    out = _combine_stage(numa, numb, skip, bias.reshape(1, HF), r)
    return (out, edge_index, edge_prob)


# speedup vs baseline: 132.4988x; 1.0715x over previous
import functools

import numpy as np
import jax
import jax.numpy as jnp
from jax import lax
from jax.experimental import pallas as pl
from jax.experimental.pallas import tpu as pltpu
from jax.experimental.pallas import tpu_sc as plsc

N = 10000
E = 320000
D = 128
H = 8
F = 16
HF = H * F
HH = HF // 2
PW = HH + 16

NW = 32
EPT = E // NW
K = 80
NCH = EPT // K
PAIRS = (NCH - 1) // 2
STRIPE = 624
ZCH = 208
TAIL = N - 16 * STRIPE
BN = 400

_RA = np.equal.outer(np.arange(HF) // F, np.arange(16)).astype(np.float32)
_RB = np.equal.outer(np.arange(16), np.arange(HF) // F).astype(np.float32)


def _dense_body(x_ref, wp0_ref, wp1_ref, ws_ref, asrc_ref, atrg_ref,
                wtp_ref, atp_ref, ra_ref, ptab0_ref, ptab1_ref, skip_ref,
                trgtab_ref, c_ref):
    xb = x_ref[...]
    dn = (((1,), (1,)), ((), ()))
    p0 = lax.dot_general(xb, wp0_ref[...], dn,
                         preferred_element_type=jnp.float32)
    p1 = lax.dot_general(xb, wp1_ref[...], dn,
                         preferred_element_type=jnp.float32)
    skip_ref[...] = lax.dot_general(xb, ws_ref[...], dn,
                                    preferred_element_type=jnp.float32)
    asrc = asrc_ref[...]
    atrg = atrg_ref[...]
    ra = ra_ref[...]
    stab = (
        jnp.dot(p0 * asrc[:, :HH], ra[:HH], preferred_element_type=jnp.float32)
        + jnp.dot(p1 * asrc[:, HH:], ra[HH:], preferred_element_type=jnp.float32))
    ptab0_ref[...] = jnp.concatenate([p0, stab], axis=-1)
    ptab1_ref[...] = jnp.concatenate([p1, stab], axis=-1)
    trgtab_ref[...] = (
        jnp.dot(p0 * atrg[:, :HH], ra[:HH], preferred_element_type=jnp.float32)
        + jnp.dot(p1 * atrg[:, HH:], ra[HH:], preferred_element_type=jnp.float32))
    c_ref[...] = jnp.dot(wtp_ref[...] * atp_ref[...], ra,
                         preferred_element_type=jnp.float32)


def _dense_stage(x, W_proj, W_skip, asrc, atrg, wtp, atp, ra):
    grid = (N // BN,)
    return pl.pallas_call(
        _dense_body,
        grid=grid,
        in_specs=[
            pl.BlockSpec((BN, D), lambda i: (i, 0)),
            pl.BlockSpec((HH, D), lambda i: (0, 0)),
            pl.BlockSpec((HH, D), lambda i: (1, 0)),
            pl.BlockSpec((HF, D), lambda i: (0, 0)),
            pl.BlockSpec((1, HF), lambda i: (0, 0)),
            pl.BlockSpec((1, HF), lambda i: (0, 0)),
            pl.BlockSpec((1, HF), lambda i: (0, 0)),
            pl.BlockSpec((1, HF), lambda i: (0, 0)),
            pl.BlockSpec((HF, 16), lambda i: (0, 0)),
        ],
        out_specs=[
            pl.BlockSpec((BN, PW), lambda i: (i, 0)),
            pl.BlockSpec((BN, PW), lambda i: (i, 0)),
            pl.BlockSpec((BN, HF), lambda i: (i, 0)),
            pl.BlockSpec((BN, 16), lambda i: (i, 0)),
            pl.BlockSpec((1, 16), lambda i: (0, 0)),
        ],
        out_shape=[
            jax.ShapeDtypeStruct((N, PW), jnp.float32),
            jax.ShapeDtypeStruct((N, PW), jnp.float32),
            jax.ShapeDtypeStruct((N, HF), jnp.float32),
            jax.ShapeDtypeStruct((N, 16), jnp.float32),
            jax.ShapeDtypeStruct((1, 16), jnp.float32),
        ],
    )(x, W_proj, W_proj, W_skip, asrc, atrg, wtp, atp, ra)


def _sc_edge_body(ptab0_hbm, ptab1_hbm, trgtab_hbm, ei_hbm, prob_hbm, c_hbm,
                  numa_hbm, numb_hbm,
                  src_v, trg_v, prob_v, b_v, p_v, w_v, c_v, z80_v,
                  acc_sh, gsem0, gsem1, ssem0, ssem1):
    cid = lax.axis_index("c")
    sid = lax.axis_index("s")
    wid = sid * 2 + cid
    base = wid * EPT

    pltpu.sync_copy(ei_hbm.at[0, pl.ds(base, EPT)], src_v)
    pltpu.sync_copy(ei_hbm.at[1, pl.ds(base, EPT)], trg_v)
    pltpu.sync_copy(prob_hbm.at[pl.ds(base, EPT)], prob_v.at[pl.ds(0, EPT)])
    pltpu.sync_copy(c_hbm, c_v)
    c_vec = c_v[0, :]

    zv = jnp.zeros((16,), jnp.float32)

    def zfill(r, _):
        for h in range(5):
            z80_v[r, pl.ds(h * 16, 16)] = zv
        return 0

    lax.fori_loop(0, ZCH, zfill, 0)

    def zero_stripes():
        for kk in range(STRIPE // ZCH):
            r0 = sid * STRIPE + kk * ZCH
            pltpu.sync_copy(z80_v, acc_sh.at[pl.ds(r0, ZCH)])

        @pl.when(sid == 15)
        def _zero_tail():
            pltpu.sync_copy(z80_v.at[pl.ds(0, TAIL)],
                            acc_sh.at[pl.ds(16 * STRIPE, TAIL)])

    zero_stripes()
    plsc.subcore_barrier()

    def run_pass(ptab_hbm, h0):
        gsems = (gsem0, gsem1)
        ssems = (ssem0, ssem1)

        def fire(j, par):
            po = par * K
            pltpu.async_copy(ptab_hbm.at[src_v.at[pl.ds(j * K, K)]],
                             p_v.at[pl.ds(po, K)], gsems[par])
            pltpu.async_copy(trgtab_hbm.at[trg_v.at[pl.ds(j * K, K)]],
                             b_v.at[pl.ds(po, K)], gsems[par])

        def wait_gathers(j, par):
            po = par * K
            pltpu.make_async_copy(ptab_hbm.at[src_v.at[pl.ds(j * K, K)]],
                                  p_v.at[pl.ds(po, K)], gsems[par]).wait()
            pltpu.make_async_copy(trgtab_hbm.at[src_v.at[pl.ds(j * K, K)]],
                                  b_v.at[pl.ds(po, K)], gsems[par]).wait()

        def wait_scatters(j, par):
            po = par * K
            pltpu.make_async_copy(
                w_v.at[pl.ds(po, K)],
                acc_sh.at[trg_v.at[pl.ds(j * K, K)]], ssems[par]).wait()

        def compute_scatter(j, par):
            po = par * K

            @plsc.parallel_loop(0, K, unroll=8)
            def edge(i):
                a = p_v[po + i, pl.ds(HH, 16)]
                bb = b_v[po + i, :]
                pv = prob_v[pl.ds(j * K + i, 16)]
                s = a + bb + pv[0] * c_vec
                s = jnp.where(s > 0.0, s, 0.2 * s)
                e = jnp.exp(s)
                w_v[po + i, pl.ds(HH, 16)] = e
                for h in range(4):
                    w_v[po + i, pl.ds(h * 16, 16)] = (
                        p_v[po + i, pl.ds(h * 16, 16)] * e[h0 + h])

            pltpu.async_copy(w_v.at[pl.ds(po, K)],
                             acc_sh.at[trg_v.at[pl.ds(j * K, K)]],
                             ssems[par], add=True)

        fire(0, 0)

        def pair(jj, _):
            for b in range(2):
                j = jj * 2 + b
                if b == 0:
                    fire(j + 1, 1)
                else:
                    @pl.when(jj < PAIRS - 1)
                    def _fire_next():
                        fire(j + 1, 0)

                wait_gathers(j, b)

                @pl.when(jj >= 1)
                def _reuse_wait():
                    wait_scatters(j, b)

                compute_scatter(j, b)
            return 0

        lax.fori_loop(0, PAIRS, pair, 0)
        fire(NCH - 1, 0)
        wait_scatters(NCH - 1, 0)
        wait_gathers(NCH - 1, 0)
        compute_scatter(NCH - 1, 0)
        wait_scatters(0, 1)
        wait_scatters(0, 0)

    def writeback(dst_hbm):
        out_base = cid * N + sid * STRIPE
        pltpu.sync_copy(acc_sh.at[pl.ds(sid * STRIPE, STRIPE)],
                        dst_hbm.at[pl.ds(out_base, STRIPE), pl.ds(0, PW)])

        @pl.when(sid == 15)
        def _tail():
            pltpu.sync_copy(
                acc_sh.at[pl.ds(16 * STRIPE, TAIL)],
                dst_hbm.at[pl.ds(cid * N + 16 * STRIPE, TAIL), pl.ds(0, PW)])

    run_pass(ptab0_hbm, 0)
    plsc.subcore_barrier()
    writeback(numa_hbm)
    zero_stripes()
    plsc.subcore_barrier()

    run_pass(ptab1_hbm, 4)
    plsc.subcore_barrier()
    writeback(numb_hbm)


def _sc_edge_stage(ptab0, ptab1, trgtab, edge_index, prob, cvec):
    mesh = plsc.VectorSubcoreMesh(core_axis_name="c", subcore_axis_name="s")
    fn = pl.kernel(
        _sc_edge_body,
        compiler_params=pltpu.CompilerParams(use_tc_tiling_on_sc=False),
        out_type=[
            jax.ShapeDtypeStruct((2 * N, D), jnp.float32),
            jax.ShapeDtypeStruct((2 * N, D), jnp.float32),
        ],
        mesh=mesh,
        scratch_types=[
            pltpu.VMEM((EPT,), jnp.int32),
            pltpu.VMEM((EPT,), jnp.int32),
            pltpu.VMEM((EPT + 16,), jnp.float32),
            pltpu.VMEM((2 * K, 16), jnp.float32),
            pltpu.VMEM((2 * K, PW), jnp.float32),
            pltpu.VMEM((2 * K, PW), jnp.float32),
            pltpu.VMEM((1, 16), jnp.float32),
            pltpu.VMEM((ZCH, PW), jnp.float32),
            pltpu.VMEM_SHARED((N, PW), jnp.float32),
            pltpu.SemaphoreType.DMA,
            pltpu.SemaphoreType.DMA,
            pltpu.SemaphoreType.DMA,
            pltpu.SemaphoreType.DMA,
        ],
    )
    return fn(ptab0, ptab1, trgtab, edge_index, prob, cvec)


def _combine_body(na0_ref, na1_ref, nb0_ref, nb1_ref,
                  skip_ref, bias_ref, r_ref, out_ref):
    na = na0_ref[...] + na1_ref[...]
    nb = nb0_ref[...] + nb1_ref[...]
    d = jnp.dot(na[:, HH:PW], r_ref[...],
                preferred_element_type=jnp.float32) + 1e-16
    num = jnp.concatenate([na[:, :HH], nb[:, :HH]], axis=-1)
    y = num / d + skip_ref[...] + bias_ref[...]
    out_ref[...] = jnp.where(y > 0.0, y, jnp.exp(jnp.minimum(y, 0.0)) - 1.0)


def _combine_stage(numa, numb, skip, bias, r):
    grid = (N // BN,)
    half = N // BN
    return pl.pallas_call(
        _combine_body,
        grid=grid,
        in_specs=[
            pl.BlockSpec((BN, D), lambda i: (i, 0)),
            pl.BlockSpec((BN, D), lambda i: (i + half, 0)),
            pl.BlockSpec((BN, D), lambda i: (i, 0)),
            pl.BlockSpec((BN, D), lambda i: (i + half, 0)),
            pl.BlockSpec((BN, HF), lambda i: (i, 0)),
            pl.BlockSpec((1, HF), lambda i: (0, 0)),
            pl.BlockSpec((16, HF), lambda i: (0, 0)),
        ],
        out_specs=pl.BlockSpec((BN, HF), lambda i: (i, 0)),
        out_shape=jax.ShapeDtypeStruct((N, HF), jnp.float32),
    )(numa, numa, numb, numb, skip, bias, r)


def kernel(x, edge_index, edge_prob, W_proj, W_tp, a_src, a_trg, a_tp,
           W_skip, bias):
    asrc = a_src.reshape(1, HF)
    atrg = a_trg.reshape(1, HF)
    wtp = W_tp.reshape(1, HF)
    atp = a_tp.reshape(1, HF)
    ra = jnp.asarray(_RA)
    r = jnp.asarray(_RB)

    ptab0, ptab1, skip, trgtab, c2 = _dense_stage(
        x, W_proj, W_skip, asrc, atrg, wtp, atp, ra)

    prob = edge_prob.reshape(E)
    numa, numb = _sc_edge_stage(ptab0, ptab1, trgtab, edge_index, prob, c2)

    out = _combine_stage(numa, numb, skip, bias.reshape(1, HF), r)
    return (out, edge_index, edge_prob)


# speedup vs baseline: 132.7192x; 1.0017x over previous
import functools

import numpy as np
import jax
import jax.numpy as jnp
from jax import lax
from jax.experimental import pallas as pl
from jax.experimental.pallas import tpu as pltpu
from jax.experimental.pallas import tpu_sc as plsc

N = 10000
E = 320000
D = 128
H = 8
F = 16
HF = H * F
HH = HF // 2
PW = HH + 16

NW = 32
EPT = E // NW
K = 80
NCH = EPT // K
PAIRS = (NCH - 1) // 2
STRIPE = 624
ZCH = 208
TAIL = N - 16 * STRIPE
BN = 400

_RA = np.equal.outer(np.arange(HF) // F, np.arange(16)).astype(np.float32)
_RB = np.equal.outer(np.arange(16), np.arange(HF) // F).astype(np.float32)


def _dense_body(x_ref, wp0_ref, wp1_ref, ws_ref, asrc_ref, atrg_ref,
                wtp_ref, atp_ref, ra_ref, ptab0_ref, ptab1_ref, skip_ref,
                trgtab_ref, c_ref):
    xb = x_ref[...]
    dn = (((1,), (1,)), ((), ()))
    p0 = lax.dot_general(xb, wp0_ref[...], dn,
                         preferred_element_type=jnp.float32)
    p1 = lax.dot_general(xb, wp1_ref[...], dn,
                         preferred_element_type=jnp.float32)
    skip_ref[...] = lax.dot_general(xb, ws_ref[...], dn,
                                    preferred_element_type=jnp.float32)
    asrc = asrc_ref[...]
    atrg = atrg_ref[...]
    ra = ra_ref[...]
    stab = (
        jnp.dot(p0 * asrc[:, :HH], ra[:HH], preferred_element_type=jnp.float32)
        + jnp.dot(p1 * asrc[:, HH:], ra[HH:], preferred_element_type=jnp.float32))
    ptab0_ref[...] = jnp.concatenate([p0, stab], axis=-1)
    ptab1_ref[...] = jnp.concatenate([p1, stab], axis=-1)
    trgtab_ref[...] = (
        jnp.dot(p0 * atrg[:, :HH], ra[:HH], preferred_element_type=jnp.float32)
        + jnp.dot(p1 * atrg[:, HH:], ra[HH:], preferred_element_type=jnp.float32))
    c_ref[...] = jnp.dot(wtp_ref[...] * atp_ref[...], ra,
                         preferred_element_type=jnp.float32)


def _dense_stage(x, W_proj, W_skip, asrc, atrg, wtp, atp, ra):
    grid = (N // BN,)
    return pl.pallas_call(
        _dense_body,
        grid=grid,
        in_specs=[
            pl.BlockSpec((BN, D), lambda i: (i, 0)),
            pl.BlockSpec((HH, D), lambda i: (0, 0)),
            pl.BlockSpec((HH, D), lambda i: (1, 0)),
            pl.BlockSpec((HF, D), lambda i: (0, 0)),
            pl.BlockSpec((1, HF), lambda i: (0, 0)),
            pl.BlockSpec((1, HF), lambda i: (0, 0)),
            pl.BlockSpec((1, HF), lambda i: (0, 0)),
            pl.BlockSpec((1, HF), lambda i: (0, 0)),
            pl.BlockSpec((HF, 16), lambda i: (0, 0)),
        ],
        out_specs=[
            pl.BlockSpec((BN, PW), lambda i: (i, 0)),
            pl.BlockSpec((BN, PW), lambda i: (i, 0)),
            pl.BlockSpec((BN, HF), lambda i: (i, 0)),
            pl.BlockSpec((BN, 16), lambda i: (i, 0)),
            pl.BlockSpec((1, 16), lambda i: (0, 0)),
        ],
        out_shape=[
            jax.ShapeDtypeStruct((N, PW), jnp.float32),
            jax.ShapeDtypeStruct((N, PW), jnp.float32),
            jax.ShapeDtypeStruct((N, HF), jnp.float32),
            jax.ShapeDtypeStruct((N, 16), jnp.float32),
            jax.ShapeDtypeStruct((1, 16), jnp.float32),
        ],
    )(x, W_proj, W_proj, W_skip, asrc, atrg, wtp, atp, ra)


def _sc_edge_body(ptab0_hbm, ptab1_hbm, trgtab_hbm, ei_hbm, prob_hbm, c_hbm,
                  numa_hbm, numb_hbm,
                  src_v, trg_v, prob_v, b_v, p_v, w_v, c_v, z80_v,
                  acc_sh, gsem0, gsem1, ssem0, ssem1):
    cid = lax.axis_index("c")
    sid = lax.axis_index("s")
    wid = sid * 2 + cid
    base = wid * EPT

    pltpu.sync_copy(ei_hbm.at[0, pl.ds(base, EPT)], src_v)
    pltpu.sync_copy(ei_hbm.at[1, pl.ds(base, EPT)], trg_v)
    pltpu.sync_copy(prob_hbm.at[pl.ds(base, EPT)], prob_v.at[pl.ds(0, EPT)])
    pltpu.sync_copy(c_hbm, c_v)
    c_vec = c_v[0, :]

    zv = jnp.zeros((16,), jnp.float32)

    def zfill(r, _):
        for h in range(5):
            z80_v[r, pl.ds(h * 16, 16)] = zv
        return 0

    lax.fori_loop(0, ZCH, zfill, 0)

    def zero_stripes():
        for kk in range(STRIPE // ZCH):
            r0 = sid * STRIPE + kk * ZCH
            pltpu.sync_copy(z80_v, acc_sh.at[pl.ds(r0, ZCH)])

        @pl.when(sid == 15)
        def _zero_tail():
            pltpu.sync_copy(z80_v.at[pl.ds(0, TAIL)],
                            acc_sh.at[pl.ds(16 * STRIPE, TAIL)])

    zero_stripes()
    plsc.subcore_barrier()

    def run_pass(ptab_hbm, h0):
        gsems = (gsem0, gsem1)
        ssems = (ssem0, ssem1)

        def fire(j, par):
            po = par * K
            pltpu.async_copy(ptab_hbm.at[src_v.at[pl.ds(j * K, K)]],
                             p_v.at[pl.ds(po, K)], gsems[par])
            pltpu.async_copy(trgtab_hbm.at[trg_v.at[pl.ds(j * K, K)]],
                             b_v.at[pl.ds(po, K)], gsems[par])

        def wait_gathers(j, par):
            po = par * K
            pltpu.make_async_copy(ptab_hbm.at[src_v.at[pl.ds(j * K, K)]],
                                  p_v.at[pl.ds(po, K)], gsems[par]).wait()
            pltpu.make_async_copy(trgtab_hbm.at[src_v.at[pl.ds(j * K, K)]],
                                  b_v.at[pl.ds(po, K)], gsems[par]).wait()

        def wait_scatters(j, par):
            po = par * K
            pltpu.make_async_copy(
                w_v.at[pl.ds(po, K)],
                acc_sh.at[trg_v.at[pl.ds(j * K, K)]], ssems[par]).wait()

        def compute_scatter(j, par):
            po = par * K

            @plsc.parallel_loop(0, K, unroll=8)
            def edge(i):
                a = p_v[po + i, pl.ds(HH, 16)]
                bb = b_v[po + i, :]
                pv = prob_v[pl.ds(j * K + i, 16)]
                s = a + bb + pv[0] * c_vec
                s = jnp.where(s > 0.0, s, 0.2 * s)
                e = jnp.exp(s)
                w_v[po + i, pl.ds(HH, 16)] = e
                for h in range(4):
                    w_v[po + i, pl.ds(h * 16, 16)] = (
                        p_v[po + i, pl.ds(h * 16, 16)] * e[h0 + h])

            pltpu.async_copy(w_v.at[pl.ds(po, K)],
                             acc_sh.at[trg_v.at[pl.ds(j * K, K)]],
                             ssems[par], add=True)

        fire(0, 0)

        def pair(jj, _):
            for b in range(2):
                j = jj * 2 + b
                if b == 0:
                    fire(j + 1, 1)
                else:
                    @pl.when(jj < PAIRS - 1)
                    def _fire_next():
                        fire(j + 1, 0)

                wait_gathers(j, b)

                @pl.when(jj >= 1)
                def _reuse_wait():
                    wait_scatters(j, b)

                compute_scatter(j, b)
            return 0

        lax.fori_loop(0, PAIRS, pair, 0)
        fire(NCH - 1, 0)
        wait_scatters(NCH - 1, 0)
        wait_gathers(NCH - 1, 0)
        compute_scatter(NCH - 1, 0)
        wait_scatters(0, 1)
        wait_scatters(0, 0)

    def writeback(dst_hbm):
        out_base = cid * N + sid * STRIPE
        pltpu.sync_copy(acc_sh.at[pl.ds(sid * STRIPE, STRIPE)],
                        dst_hbm.at[pl.ds(out_base, STRIPE), pl.ds(0, PW)])

        @pl.when(sid == 15)
        def _tail():
            pltpu.sync_copy(
                acc_sh.at[pl.ds(16 * STRIPE, TAIL)],
                dst_hbm.at[pl.ds(cid * N + 16 * STRIPE, TAIL), pl.ds(0, PW)])

    run_pass(ptab0_hbm, 0)
    plsc.subcore_barrier()
    writeback(numa_hbm)
    zero_stripes()
    plsc.subcore_barrier()

    run_pass(ptab1_hbm, 4)
    plsc.subcore_barrier()
    writeback(numb_hbm)


def _sc_edge_stage(ptab0, ptab1, trgtab, edge_index, prob, cvec):
    mesh = plsc.VectorSubcoreMesh(core_axis_name="c", subcore_axis_name="s")
    fn = pl.kernel(
        _sc_edge_body,
        compiler_params=pltpu.CompilerParams(use_tc_tiling_on_sc=False),
        out_type=[
            jax.ShapeDtypeStruct((2 * N, D), jnp.float32),
            jax.ShapeDtypeStruct((2 * N, D), jnp.float32),
        ],
        mesh=mesh,
        scratch_types=[
            pltpu.VMEM((EPT,), jnp.int32),
            pltpu.VMEM((EPT,), jnp.int32),
            pltpu.VMEM((EPT + 16,), jnp.float32),
            pltpu.VMEM((2 * K, 16), jnp.float32),
            pltpu.VMEM((2 * K, PW), jnp.float32),
            pltpu.VMEM((2 * K, PW), jnp.float32),
            pltpu.VMEM((1, 16), jnp.float32),
            pltpu.VMEM((ZCH, PW), jnp.float32),
            pltpu.VMEM_SHARED((N, PW), jnp.float32),
            pltpu.SemaphoreType.DMA,
            pltpu.SemaphoreType.DMA,
            pltpu.SemaphoreType.DMA,
            pltpu.SemaphoreType.DMA,
        ],
    )
    return fn(ptab0, ptab1, trgtab, edge_index, prob, cvec)


def _combine_body(na0_ref, na1_ref, nb0_ref, nb1_ref,
                  skip_ref, bias_ref, r_ref, out_ref):
    na = na0_ref[...] + na1_ref[...]
    nb = nb0_ref[...] + nb1_ref[...]
    d = jnp.dot(na[:, HH:PW], r_ref[...],
                preferred_element_type=jnp.float32) + 1e-16
    num = jnp.concatenate([na[:, :HH], nb[:, :HH]], axis=-1)
    y = num / d + skip_ref[...] + bias_ref[...]
    out_ref[...] = jnp.where(y > 0.0, y, jnp.exp(jnp.minimum(y, 0.0)) - 1.0)


def _combine_stage(numa, numb, skip, bias, r):
    grid = (N // BN,)
    half = N // BN
    return pl.pallas_call(
        _combine_body,
        grid=grid,
        in_specs=[
            pl.BlockSpec((BN, D), lambda i: (i, 0)),
            pl.BlockSpec((BN, D), lambda i: (i + half, 0)),
            pl.BlockSpec((BN, D), lambda i: (i, 0)),
            pl.BlockSpec((BN, D), lambda i: (i + half, 0)),
            pl.BlockSpec((BN, HF), lambda i: (i, 0)),
            pl.BlockSpec((1, HF), lambda i: (0, 0)),
            pl.BlockSpec((16, HF), lambda i: (0, 0)),
        ],
        out_specs=pl.BlockSpec((BN, HF), lambda i: (i, 0)),
        out_shape=jax.ShapeDtypeStruct((N, HF), jnp.float32),
    )(numa, numa, numb, numb, skip, bias, r)


def kernel(x, edge_index, edge_prob, W_proj, W_tp, a_src, a_trg, a_tp,
           W_skip, bias):
    asrc = a_src.reshape(1, HF)
    atrg = a_trg.reshape(1, HF)
    wtp = W_tp.reshape(1, HF)
    atp = a_tp.reshape(1, HF)
    ra = jnp.asarray(_RA)
    r = jnp.asarray(_RB)

    ptab0, ptab1, skip, trgtab, c2 = _dense_stage(
        x, W_proj, W_skip, asrc, atrg, wtp, atp, ra)

    prob = lax.squeeze(edge_prob, dimensions=(1,))
    numa, numb = _sc_edge_stage(ptab0, ptab1, trgtab, edge_index, prob, c2)

    out = _combine_stage(numa, numb, skip, bias.reshape(1, HF), r)
    return (out, edge_index, edge_prob)
